# Initial kernel scaffold; baseline (speedup 1.0000x reference)
import jax
import jax.numpy as jnp
from jax.experimental import pallas as pl


def kernel(features_per_scale_0, features_per_scale_1, features_per_scale_2, scale_embeddings, patch_embeddings):
    raise NotImplementedError("write your pallas kernel here")


# speedup vs baseline: 2.0461x; 2.0461x over previous
import jax
import jax.numpy as jnp
from jax.experimental import pallas as pl

_D = 768
_ROWS = 64
_NB0, _NB1, _NB2 = 16, 4, 1
_NTOT = _NB0 + _NB1 + _NB2


def _body(f0_ref, f1_ref, f2_ref, se_ref, pe_ref, out_ref):
    j = pl.program_id(0)
    pe = pe_ref[...]

    @pl.when(j < _NB0)
    def _():
        out_ref[...] = f0_ref[...] + (se_ref[0, :][None, None, :] + pe)

    @pl.when(jnp.logical_and(j >= _NB0, j < _NB0 + _NB1))
    def _():
        out_ref[...] = f1_ref[...] + (se_ref[1, :][None, None, :] + pe)

    @pl.when(j >= _NB0 + _NB1)
    def _():
        out_ref[...] = f2_ref[...] + (se_ref[2, :][None, None, :] + pe)


def _scale_of(j):
    return jnp.where(j < _NB0, 0, jnp.where(j < _NB0 + _NB1, 1, 2))


def _rowblock_of(j):
    return jnp.where(j < _NB0, j,
                     jnp.where(j < _NB0 + _NB1, j - _NB0, j - _NB0 - _NB1))


def kernel(features_per_scale_0, features_per_scale_1, features_per_scale_2,
           scale_embeddings, patch_embeddings):
    B = features_per_scale_0.shape[0]
    n_out = (_NTOT) * _ROWS

    return pl.pallas_call(
        _body,
        grid=(_NTOT,),
        in_specs=[
            pl.BlockSpec((B, _ROWS, _D), lambda j: (0, jnp.minimum(j, _NB0 - 1), 0)),
            pl.BlockSpec((B, _ROWS, _D), lambda j: (0, jnp.clip(j - _NB0, 0, _NB1 - 1), 0)),
            pl.BlockSpec((B, _ROWS, _D), lambda j: (0, 0, 0)),
            pl.BlockSpec((3, _D), lambda j: (0, 0)),
            pl.BlockSpec((1, _ROWS, _D), lambda j: (_scale_of(j), _rowblock_of(j), 0)),
        ],
        out_specs=pl.BlockSpec((B, _ROWS, _D), lambda j: (0, j, 0)),
        out_shape=jax.ShapeDtypeStruct((B, n_out, _D), jnp.float32),
    )(features_per_scale_0, features_per_scale_1, features_per_scale_2,
      scale_embeddings, patch_embeddings)


# speedup vs baseline: 2.0499x; 1.0019x over previous
import jax
import jax.numpy as jnp
from jax.experimental import pallas as pl
from jax.experimental.pallas import tpu as pltpu

_D = 768
_ROWS = 64
_NB0, _NB1, _NB2 = 16, 4, 1
_NTOT = _NB0 + _NB1 + _NB2


def _body(f0_ref, f1_ref, f2_ref, se_ref, pe_ref, out_ref):
    j = pl.program_id(0)
    pe = pe_ref[...]

    @pl.when(j < _NB0)
    def _():
        out_ref[...] = f0_ref[...] + (se_ref[0, :][None, None, :] + pe)

    @pl.when(jnp.logical_and(j >= _NB0, j < _NB0 + _NB1))
    def _():
        out_ref[...] = f1_ref[...] + (se_ref[1, :][None, None, :] + pe)

    @pl.when(j >= _NB0 + _NB1)
    def _():
        out_ref[...] = f2_ref[...] + (se_ref[2, :][None, None, :] + pe)


def _scale_of(j):
    return jnp.where(j < _NB0, 0, jnp.where(j < _NB0 + _NB1, 1, 2))


def _rowblock_of(j):
    return jnp.where(j < _NB0, j,
                     jnp.where(j < _NB0 + _NB1, j - _NB0, j - _NB0 - _NB1))


def kernel(features_per_scale_0, features_per_scale_1, features_per_scale_2,
           scale_embeddings, patch_embeddings):
    B = features_per_scale_0.shape[0]
    n_out = (_NTOT) * _ROWS

    return pl.pallas_call(
        _body,
        grid=(_NTOT,),
        in_specs=[
            pl.BlockSpec((B, _ROWS, _D), lambda j: (0, jnp.minimum(j, _NB0 - 1), 0)),
            pl.BlockSpec((B, _ROWS, _D), lambda j: (0, jnp.clip(j - _NB0, 0, _NB1 - 1), 0)),
            pl.BlockSpec((B, _ROWS, _D), lambda j: (0, 0, 0)),
            pl.BlockSpec((3, _D), lambda j: (0, 0)),
            pl.BlockSpec((1, _ROWS, _D), lambda j: (_scale_of(j), _rowblock_of(j), 0)),
        ],
        out_specs=pl.BlockSpec((B, _ROWS, _D), lambda j: (0, j, 0)),
        out_shape=jax.ShapeDtypeStruct((B, n_out, _D), jnp.float32),
        compiler_params=pltpu.CompilerParams(
            dimension_semantics=("parallel",)),
    )(features_per_scale_0, features_per_scale_1, features_per_scale_2,
      scale_embeddings, patch_embeddings)
